# Initial kernel scaffold; baseline (speedup 1.0000x reference)
import jax
import jax.numpy as jnp
from jax.experimental import pallas as pl


def kernel(coords, features, W, b):
    raise NotImplementedError("write your pallas kernel here")


# speedup vs baseline: 4.2886x; 4.2886x over previous
import functools

import jax
import jax.numpy as jnp
from jax import lax
from jax.experimental import pallas as pl
from jax.experimental.pallas import tpu as pltpu
from jax.experimental.pallas import tpu_sc as plsc

N = 8192
D = 32
K = 16
L = 16
NC = 2
NS = 16
NW = NC * NS
QPW = N // NW
NCHUNK = N // L
CAND_CAP = 1024

_INF = float("inf")


def _sc_knn_body(cx_hbm, cy_hbm, cz_hbm, outx_hbm, outy_hbm, outz_hbm,
                 xs, ys, zs, dbuf, cand_d2, cand_idx, stgx, stgy, stgz):
    wid = lax.axis_index("s") * NC + lax.axis_index("c")
    base = wid * QPW

    pltpu.sync_copy(cx_hbm, xs)
    pltpu.sync_copy(cy_hbm, ys)
    pltpu.sync_copy(cz_hbm, zs)

    iota = lax.iota(jnp.int32, L)

    def per_query(qi, _):
        q = base + qi
        qvec = jnp.full((L,), q, dtype=jnp.int32)
        qx = plsc.load_gather(xs, [qvec])
        qy = plsc.load_gather(ys, [qvec])
        qz = plsc.load_gather(zs, [qvec])

        def pass_a(c, minv):
            off = c * L
            rx = xs[pl.ds(off, L)] - qx
            ry = ys[pl.ds(off, L)] - qy
            rz = zs[pl.ds(off, L)] - qz
            d2 = rx * rx + ry * ry + rz * rz
            dbuf[pl.ds(off, L)] = d2
            return jnp.minimum(minv, d2)

        minv = lax.fori_loop(0, NCHUNK, pass_a, jnp.full((L,), _INF))
        thresh = jnp.max(minv)

        def pass_b(c, cursor):
            off = c * L
            d2 = dbuf[pl.ds(off, L)]
            m = d2 <= thresh
            plsc.store_compressed(cand_d2.at[pl.ds(cursor, L)], d2, mask=m)
            plsc.store_compressed(cand_idx.at[pl.ds(cursor, L)], iota + off,
                                  mask=m)
            cnt = jnp.max(plsc.all_reduce_population_count(m))
            return jnp.minimum(cursor + cnt, CAND_CAP - L)

        cursor = lax.fori_loop(0, NCHUNK, pass_b, jnp.int32(0))
        cand_d2[pl.ds(cursor, L)] = jnp.full((L,), _INF)
        cand_idx[pl.ds(cursor, L)] = qvec

        def pass_c(j, carry):
            rv, ri = carry
            sv = cand_d2[pl.ds(j * L, L)]
            si = cand_idx[pl.ds(j * L, L)]
            sv, si = plsc.sort_key_val(sv, si)
            rb = lax.rev(sv, (0,))
            rbi = lax.rev(si, (0,))
            take = rv <= rb
            mv = jnp.where(take, rv, rb)
            mi = jnp.where(take, ri, rbi)
            return tuple(plsc.sort_key_val(mv, mi))

        nch = (cursor + L - 1) // L
        rv0 = jnp.full((L,), _INF)
        ri0 = jnp.zeros((L,), jnp.int32)
        _, ri = lax.fori_loop(0, nch, pass_c, (rv0, ri0))

        stgx[pl.ds(qi * L, L)] = plsc.load_gather(xs, [ri])
        stgy[pl.ds(qi * L, L)] = plsc.load_gather(ys, [ri])
        stgz[pl.ds(qi * L, L)] = plsc.load_gather(zs, [ri])
        return 0

    lax.fori_loop(0, QPW, per_query, 0)

    row = base * L
    pltpu.sync_copy(stgx, outx_hbm.at[pl.ds(row, QPW * L)])
    pltpu.sync_copy(stgy, outy_hbm.at[pl.ds(row, QPW * L)])
    pltpu.sync_copy(stgz, outz_hbm.at[pl.ds(row, QPW * L)])


_sc_knn = pl.kernel(
    _sc_knn_body,
    out_type=[jax.ShapeDtypeStruct((N * K,), jnp.float32)] * 3,
    mesh=plsc.VectorSubcoreMesh(core_axis_name="c", subcore_axis_name="s"),
    compiler_params=pltpu.CompilerParams(needs_layout_passes=False),
    scratch_types=[
        pltpu.VMEM((N,), jnp.float32),
        pltpu.VMEM((N,), jnp.float32),
        pltpu.VMEM((N,), jnp.float32),
        pltpu.VMEM((N,), jnp.float32),
        pltpu.VMEM((CAND_CAP,), jnp.float32),
        pltpu.VMEM((CAND_CAP,), jnp.int32),
        pltpu.VMEM((QPW * L,), jnp.float32),
        pltpu.VMEM((QPW * L,), jnp.float32),
        pltpu.VMEM((QPW * L,), jnp.float32),
    ],
)


BQ = 512


def _tc_body(coords_ref, feat_ref, nx_ref, ny_ref, nz_ref, w_ref, b_ref,
             out_ref):
    nx = nx_ref[...]
    ny = ny_ref[...]
    nz = nz_ref[...]
    ox = coords_ref[:, 0:1]
    oy = coords_ref[:, 1:2]
    oz = coords_ref[:, 2:3]
    rx = ox - nx
    ry = oy - ny
    rz = oz - nz
    sq = rx * rx + ry * ry + rz * rz
    safe = jnp.where(sq > 0, sq, 1.0)
    dist = jnp.where(sq > 0, jnp.sqrt(safe), 0.0)

    oxb = jnp.broadcast_to(ox, (BQ, K))
    oyb = jnp.broadcast_to(oy, (BQ, K))
    ozb = jnp.broadcast_to(oz, (BQ, K))
    feats = (oxb, oyb, ozb, nx, ny, nz, rx, ry, rz, dist)
    mlp = jnp.broadcast_to(b_ref[0, :][None, None, :], (BQ, K, D // 2))
    for f, x in enumerate(feats):
        mlp = mlp + x[:, :, None] * w_ref[f, :][None, None, :]
    gathered = jnp.broadcast_to(feat_ref[...][:, None, :], (BQ, K, D))
    out_ref[...] = jnp.concatenate([mlp, gathered], axis=-1)


_tc_assemble = pl.pallas_call(
    _tc_body,
    grid=(N // BQ,),
    in_specs=[
        pl.BlockSpec((BQ, 3), lambda i: (i, 0)),
        pl.BlockSpec((BQ, D), lambda i: (i, 0)),
        pl.BlockSpec((BQ, K), lambda i: (i, 0)),
        pl.BlockSpec((BQ, K), lambda i: (i, 0)),
        pl.BlockSpec((BQ, K), lambda i: (i, 0)),
        pl.BlockSpec((10, D // 2), lambda i: (0, 0)),
        pl.BlockSpec((1, D // 2), lambda i: (0, 0)),
    ],
    out_specs=pl.BlockSpec((BQ, K, 3 * D // 2), lambda i: (i, 0, 0)),
    out_shape=jax.ShapeDtypeStruct((N, K, 3 * D // 2), jnp.float32),
)


def kernel(coords, features, W, b):
    coords_t = coords.T.reshape(3, N)
    nbx, nby, nbz = _sc_knn(coords_t[0], coords_t[1], coords_t[2])
    out = _tc_assemble(coords, features,
                       nbx.reshape(N, K), nby.reshape(N, K),
                       nbz.reshape(N, K), W, b.reshape(1, D // 2))
    return out
